# Initial kernel scaffold; baseline (speedup 1.0000x reference)
import jax
import jax.numpy as jnp
from jax.experimental import pallas as pl


def kernel(x_state_enc, x_enc, x_mark_enc, x_state_dec, x_dec, x_mark_dec, params):
    raise NotImplementedError("write your pallas kernel here")


# speedup vs baseline: 6.5391x; 6.5391x over previous
import functools
import math

import numpy as np
import jax
import jax.numpy as jnp
from jax import lax
from jax.experimental import pallas as pl
from jax.experimental.pallas import tpu as pltpu
from jax.experimental.pallas import tpu_sc as plsc

L_ENC = 2048
L_DEC = 1024
PRED_LEN = 512
DM = 1024
H = 16
DH = 64
D_FF = 2048
FACTOR = 5
U_PAD = 40
NEG = -1e30

_INTERPRET = False


def _u(L):
    return min(int(FACTOR * math.ceil(math.log(L))), L)


def _count_matrix(seed, L_Q, L_K):
    U_part = _u(L_K)
    rs = np.random.RandomState(seed)
    idx = rs.randint(0, L_K, size=(L_Q, U_part))
    A = np.zeros((L_Q, L_K), np.int8)
    np.add.at(A, (np.arange(L_Q)[:, None], idx), 1)
    return A


_A100 = _count_matrix(100, L_ENC, L_ENC)
_A101 = _count_matrix(101, L_ENC, L_ENC)
_A200 = _count_matrix(200, L_DEC, L_DEC)
_A201 = _count_matrix(201, L_DEC, L_ENC)
_TRIL = np.tril(np.ones((L_DEC, L_DEC), np.float32))


def _ln_rows(x, g, b):
    m = jnp.mean(x, axis=-1, keepdims=True)
    d = x - m
    v = jnp.mean(d * d, axis=-1, keepdims=True)
    return d * jax.lax.rsqrt(v + 1e-5) * g + b


def _mm(x, w, bias=None, act=None, res=None, ln=None, pre_ln=None, bm=256):
    M, K = x.shape
    _, N = w.shape
    grid = (M // bm,)
    args = [x, w]
    specs = [pl.BlockSpec((bm, K), lambda i: (i, 0)),
             pl.BlockSpec((K, N), lambda i: (0, 0))]
    if pre_ln is not None:
        args += [pre_ln[0].reshape(1, K), pre_ln[1].reshape(1, K)]
        specs += [pl.BlockSpec((1, K), lambda i: (0, 0))] * 2
    if bias is not None:
        args.append(bias.reshape(1, N))
        specs.append(pl.BlockSpec((1, N), lambda i: (0, 0)))
    if res is not None:
        args.append(res)
        specs.append(pl.BlockSpec((bm, N), lambda i: (i, 0)))
    if ln is not None:
        args += [ln[0].reshape(1, N), ln[1].reshape(1, N)]
        specs += [pl.BlockSpec((1, N), lambda i: (0, 0))] * 2

    def body(*refs):
        it = iter(refs)
        xv = next(it)[...]
        wv = next(it)[...]
        if pre_ln is not None:
            g0 = next(it)[...]
            b0 = next(it)[...]
            xv = _ln_rows(xv, g0, b0)
        y = jnp.dot(xv, wv, preferred_element_type=jnp.float32)
        if bias is not None:
            y = y + next(it)[...]
        if act == "gelu":
            y = jax.nn.gelu(y, approximate=True)
        if res is not None:
            y = y + next(it)[...]
        if ln is not None:
            g1 = next(it)[...]
            b1 = next(it)[...]
            y = _ln_rows(y, g1, b1)
        out = next(it)
        out[...] = y

    return pl.pallas_call(
        body, grid=grid, in_specs=specs,
        out_specs=pl.BlockSpec((bm, N), lambda i: (i, 0)),
        out_shape=jax.ShapeDtypeStruct((M, N), jnp.float32),
        interpret=_INTERPRET,
    )(*args)


def _prob_m(q_arr, k_arr, q_cb, k_cb, A, bq=256):
    L_Q, L_K = A.shape

    def body(q_ref, k_ref, a_ref, o_ref):
        af = a_ref[...].astype(jnp.float32)
        mask = af > 0
        q = q_ref[...]
        k = k_ref[...]
        cols = []
        for h in range(H):
            qh = q[:, h * DH:(h + 1) * DH]
            kh = k[:, h * DH:(h + 1) * DH]
            S = lax.dot_general(qh, kh, (((1,), (1,)), ((), ())),
                                preferred_element_type=jnp.float32)
            mx = jnp.max(jnp.where(mask, S, NEG), axis=1, keepdims=True)
            sm = jnp.sum(S * af, axis=1, keepdims=True)
            cols.append(mx - sm * (1.0 / L_K))
        o_ref[...] = jnp.concatenate(cols, axis=1)

    return pl.pallas_call(
        body, grid=(L_Q // bq,),
        in_specs=[
            pl.BlockSpec((bq, DM), lambda i: (i, q_cb)),
            pl.BlockSpec((L_K, DM), lambda i: (0, k_cb)),
            pl.BlockSpec((bq, L_K), lambda i: (i, 0)),
        ],
        out_specs=pl.BlockSpec((bq, H), lambda i: (i, 0)),
        out_shape=jax.ShapeDtypeStruct((L_Q, H), jnp.float32),
        interpret=_INTERPRET,
    )(q_arr, k_arr, jnp.asarray(A))


def _topk(M_arr, u):
    L = M_arr.shape[0]

    def body(m_ref, o_ref):
        m = m_ref[...]
        i0 = lax.broadcasted_iota(jnp.int32, (L, H), 0)
        colo = lax.broadcasted_iota(jnp.int32, (H, 128), 1)
        d0 = lax.broadcasted_iota(jnp.int32, (H, H), 0)
        d1 = lax.broadcasted_iota(jnp.int32, (H, H), 1)
        acc = jnp.full((H, 128), L, jnp.int32)
        for t in range(u):
            mx = jnp.max(m, axis=0, keepdims=True)
            am = jnp.min(jnp.where(m == mx, i0, L), axis=0, keepdims=True)
            amT = jnp.sum(jnp.where(d0 == d1,
                                    jnp.broadcast_to(am, (H, H)), 0),
                          axis=1, keepdims=True)
            acc = jnp.where(colo == t, jnp.broadcast_to(amT, (H, 128)), acc)
            m = jnp.where(i0 == am, NEG, m)
        o_ref[...] = acc

    return pl.pallas_call(
        body, grid=(1,),
        in_specs=[pl.BlockSpec((L, H), lambda i: (0, 0))],
        out_specs=pl.BlockSpec((H, 128), lambda i: (0, 0)),
        out_shape=jax.ShapeDtypeStruct((H, 128), jnp.int32),
        interpret=_INTERPRET,
    )(M_arr)


def _sparse_update(q_arr, kv_arr, q_cb, k_cb, v_cb, mtop3, causal):
    L_Q = q_arr.shape[0]
    L_K = kv_arr.shape[0]
    scale = 1.0 / math.sqrt(DH)

    def body(q_ref, k_ref, v_ref, mt_ref, u_ref, mv_ref):
        q2 = q_ref[...]
        k2 = k_ref[...]
        v2 = v_ref[...]
        mt_all = mt_ref[...]
        mvs = []
        for sub in range(2):
            q = q2[:, sub * DH:(sub + 1) * DH]
            k = k2[:, sub * DH:(sub + 1) * DH]
            v = v2[:, sub * DH:(sub + 1) * DH]
            mtu = mt_all[sub][:, :U_PAD]
            i0 = lax.broadcasted_iota(jnp.int32, (U_PAD, U_PAD), 0)
            i1 = lax.broadcasted_iota(jnp.int32, (U_PAD, U_PAD), 1)
            mt_col = jnp.sum(
                jnp.where(i0 == i1,
                          jnp.broadcast_to(mtu, (U_PAD, U_PAD)), 0),
                axis=1, keepdims=True)
            qcol = lax.broadcasted_iota(jnp.int32, (U_PAD, L_Q), 1)
            oh = (mt_col == qcol).astype(jnp.float32)
            q_red = jnp.dot(oh, q, preferred_element_type=jnp.float32)
            s = lax.dot_general(q_red, k, (((1,), (1,)), ((), ())),
                                preferred_element_type=jnp.float32) * scale
            if causal:
                kcol = lax.broadcasted_iota(jnp.int32, (U_PAD, L_K), 1)
                s = jnp.where(kcol > mt_col, NEG, s)
            mx = jnp.max(s, axis=1, keepdims=True)
            e = jnp.exp(s - mx)
            a = e / jnp.sum(e, axis=1, keepdims=True)
            u_ref[sub] = jnp.dot(a, v, preferred_element_type=jnp.float32)
            mvs.append(jnp.mean(v, axis=0, keepdims=True))
        mv_ref[...] = jnp.concatenate(mvs, axis=1)

    return pl.pallas_call(
        body, grid=(H // 2,),
        in_specs=[
            pl.BlockSpec((L_Q, 128), lambda h: (0, q_cb + h)),
            pl.BlockSpec((L_K, 128), lambda h: (0, k_cb + h)),
            pl.BlockSpec((L_K, 128), lambda h: (0, v_cb + h)),
            pl.BlockSpec((2, 1, 128), lambda h: (h, 0, 0)),
        ],
        out_specs=(
            pl.BlockSpec((2, U_PAD, DH), lambda h: (h, 0, 0)),
            pl.BlockSpec((1, 128), lambda h: (0, h)),
        ),
        out_shape=(
            jax.ShapeDtypeStruct((H, U_PAD, DH), jnp.float32),
            jax.ShapeDtypeStruct((1, DM), jnp.float32),
        ),
        interpret=_INTERPRET,
    )(q_arr, kv_arr, kv_arr, mtop3)


def _assemble(upd, mtop3, wo, bo, xres, g, b, L_out, row_off=0,
              meanv=None, v_arr=None, v_cb=0, tril=None, bl=256):
    cumsum = tril is not None
    off_blk = row_off // bl

    args = [upd, mtop3]
    specs = [pl.BlockSpec((H, U_PAD, DH), lambda i: (0, 0, 0)),
             pl.BlockSpec((H, 1, 128), lambda i: (0, 0, 0))]
    if cumsum:
        L_K = v_arr.shape[0]
        args += [jnp.asarray(tril), v_arr]
        specs += [pl.BlockSpec((bl, L_K), lambda i: (i, 0)),
                  pl.BlockSpec((L_K, DM), lambda i: (0, v_cb))]
    else:
        args.append(meanv)
        specs.append(pl.BlockSpec((1, DM), lambda i: (0, 0)))
    args += [wo, bo.reshape(1, DM), xres,
             g.reshape(1, DM), b.reshape(1, DM)]
    specs += [pl.BlockSpec((DM, DM), lambda i: (0, 0)),
              pl.BlockSpec((1, DM), lambda i: (0, 0)),
              pl.BlockSpec((bl, DM), lambda i: (i + off_blk, 0)),
              pl.BlockSpec((1, DM), lambda i: (0, 0)),
              pl.BlockSpec((1, DM), lambda i: (0, 0))]

    def body(*refs):
        it = iter(refs)
        upd_all = next(it)[...]
        mt_all = next(it)[...]
        if cumsum:
            trl = next(it)[...]
            vv = next(it)[...]
            base = jnp.dot(trl, vv, preferred_element_type=jnp.float32)
        else:
            base = jnp.broadcast_to(next(it)[...], (bl, DM))
        wo_v = next(it)[...]
        bo_v = next(it)[...]
        xr = next(it)[...]
        g_v = next(it)[...]
        b_v = next(it)[...]
        out = next(it)
        rid = (lax.broadcasted_iota(jnp.int32, (bl, 1), 0)
               + pl.program_id(0) * bl + row_off)
        cols = []
        for h in range(H):
            mtu = mt_all[h][:, :U_PAD]
            ohT = (rid == mtu).astype(jnp.float32)
            contrib = jnp.dot(ohT, upd_all[h],
                              preferred_element_type=jnp.float32)
            selm = jnp.max(ohT, axis=1, keepdims=True)
            cols.append(jnp.where(selm > 0, contrib,
                                  base[:, h * DH:(h + 1) * DH]))
        ctx = jnp.concatenate(cols, axis=1)
        y = jnp.dot(ctx, wo_v, preferred_element_type=jnp.float32)
        out[...] = _ln_rows(y + bo_v + xr, g_v, b_v)

    return pl.pallas_call(
        body, grid=(L_out // bl,),
        in_specs=specs,
        out_specs=pl.BlockSpec((bl, DM), lambda i: (i, 0)),
        out_shape=jax.ShapeDtypeStruct((L_out, DM), jnp.float32),
        interpret=_INTERPRET,
    )(*args)


def _sc_gather(table, idx):
    B = idx.shape[0]
    D = table.shape[1]
    NW = 32
    b_per_w = B // NW
    mesh = plsc.VectorSubcoreMesh(core_axis_name="c", subcore_axis_name="s")

    @functools.partial(
        pl.kernel, mesh=mesh,
        out_type=jax.ShapeDtypeStruct((B, D), jnp.float32),
        scratch_types=[
            pltpu.VMEM((b_per_w,), jnp.int32),
            pltpu.VMEM((b_per_w, D), jnp.float32),
            pltpu.SemaphoreType.DMA,
        ],
    )
    def k(table_hbm, idx_hbm, out_hbm, idx_v, rows_v, sem):
        wid = lax.axis_index("s") * 2 + lax.axis_index("c")
        base = wid * b_per_w
        pltpu.sync_copy(idx_hbm.at[pl.ds(base, b_per_w)], idx_v)
        pltpu.async_copy(table_hbm.at[idx_v], rows_v, sem).wait()
        pltpu.sync_copy(rows_v, out_hbm.at[pl.ds(base, b_per_w)])

    return k(table, idx)


def _token_state_embed(x, conv_w, state_rows):
    L = x.shape[0]
    xw = jnp.concatenate(
        [jnp.roll(x, 1, axis=0), x, jnp.roll(x, -1, axis=0)], axis=1)
    xw = jnp.pad(xw, ((0, 0), (0, 64 - xw.shape[1])))
    wc = jnp.concatenate([conv_w[0], conv_w[1], conv_w[2]], axis=0)
    wc = jnp.pad(wc, ((0, 64 - wc.shape[0]), (0, 0)))
    return _mm(xw, wc, res=state_rows)


def _qkv(p, x, names=("Wq", "Wk", "Wv"), bnames=("bq", "bk", "bv")):
    w = jnp.concatenate([p[n] for n in names], axis=1)
    bb = jnp.concatenate([p[n] for n in bnames], axis=0)
    return _mm(x, w, bias=bb)


def _enc_layer(p, x, A):
    ap = p["attn"]
    qkv = _qkv(ap, x)
    Mv = _prob_m(qkv, qkv, 0, 1, A)
    mtop = _topk(Mv, U_PAD).reshape(H, 1, 128)
    upd, meanv = _sparse_update(qkv, qkv, 0, 8, 16, mtop, causal=False)
    x = _assemble(upd, mtop, ap["Wo"], ap["bo"], x, p["ln1_g"], p["ln1_b"],
                  L_out=x.shape[0], meanv=meanv)
    y = _mm(x, p["W1"], bias=p["b1"], act="gelu")
    return _mm(y, p["W2"], bias=p["b2"], res=x, ln=(p["ln2_g"], p["ln2_b"]))


def kernel(x_state_enc, x_enc, x_mark_enc, x_state_dec, x_dec, x_mark_dec,
           params):
    p = params
    xe = x_enc[0]
    xd = x_dec[0]
    se = _sc_gather(p["enc_state"],
                    x_state_enc.reshape(L_ENC).astype(jnp.int32))
    sd = _sc_gather(p["dec_state"],
                    x_state_dec.reshape(L_DEC).astype(jnp.int32))
    enc = _token_state_embed(xe, p["enc_conv"], se)
    enc = _enc_layer(p["enc_layers"][0], enc, _A100)
    enc = _enc_layer(p["enc_layers"][1], enc, _A101)

    dec = _token_state_embed(xd, p["dec_conv"], sd)
    dp = p["dec_layers"][0]
    sa = dp["self_attn"]
    qkv = _qkv(sa, dec)
    u_dec = _u(L_DEC)
    Mv = _prob_m(qkv, qkv, 0, 1, _A200)
    mtop = _topk(Mv, u_dec).reshape(H, 1, 128)
    upd, _ = _sparse_update(qkv, qkv, 0, 8, 16, mtop, causal=True)
    x = _assemble(upd, mtop, sa["Wo"], sa["bo"], dec,
                  dp["ln1_g"], dp["ln1_b"], L_out=L_DEC,
                  v_arr=qkv, v_cb=2, tril=_TRIL)

    ca = dp["cross_attn"]
    Q = _mm(x, ca["Wq"], bias=ca["bq"])
    KV = _mm(enc, jnp.concatenate([ca["Wk"], ca["Wv"]], axis=1),
             bias=jnp.concatenate([ca["bk"], ca["bv"]], axis=0),
             pre_ln=(p["enc_norm_g"], p["enc_norm_b"]))
    Mv = _prob_m(Q, KV, 0, 0, _A201)
    mtop = _topk(Mv, u_dec).reshape(H, 1, 128)
    upd, meanv = _sparse_update(Q, KV, 0, 0, 8, mtop, causal=False)
    x2 = _assemble(upd, mtop, ca["Wo"], ca["bo"], x,
                   dp["ln2_g"], dp["ln2_b"], L_out=PRED_LEN, row_off=512,
                   meanv=meanv)
    y = _mm(x2, dp["W1"], bias=dp["b1"], act="gelu")
    x3 = _mm(y, dp["W2"], bias=dp["b2"], res=x2,
             ln=(dp["ln3_g"], dp["ln3_b"]))
    wp = jnp.pad(p["proj_W"], ((0, 0), (0, 128 - p["proj_W"].shape[1])))
    bp = jnp.pad(p["proj_b"], (0, 128 - p["proj_b"].shape[0]))
    out = _mm(x3, wp, bias=bp, pre_ln=(p["dec_norm_g"], p["dec_norm_b"]),
              bm=256)
    return out[:, :21].reshape(1, PRED_LEN, 21)
